# baseline (speedup 1.0000x reference)
import functools
import jax
import jax.numpy as jnp
from jax import lax
from jax.experimental import pallas as pl
from jax.experimental.pallas import tpu as pltpu
from jax.experimental.pallas import tpu_sc as plsc


def _mm_bias_kernel(a_ref, w_ref, b_ref, o_ref, *, relu):
    acc = jnp.dot(a_ref[...], w_ref[...], preferred_element_type=jnp.float32)
    acc = acc + b_ref[...]
    if relu:
        acc = jnp.maximum(acc, 0.0)
    o_ref[...] = acc


def _mm_bias(a, w, b, relu, bm, bn):
    M, K = a.shape
    _, N = w.shape
    return pl.pallas_call(
        functools.partial(_mm_bias_kernel, relu=relu),
        grid=(N // bn, M // bm),
        in_specs=[
            pl.BlockSpec((bm, K), lambda n, m: (m, 0)),
            pl.BlockSpec((K, bn), lambda n, m: (0, n)),
            pl.BlockSpec((1, bn), lambda n, m: (0, n)),
        ],
        out_specs=pl.BlockSpec((bm, bn), lambda n, m: (m, n)),
        out_shape=jax.ShapeDtypeStruct((M, N), jnp.float32),
    )(a, w, b.reshape(1, N))


def _h2_logits_kernel(h1_ref, w2_ref, b2_ref, w3_ref, b3_ref, o_ref):
    i = pl.program_id(0)

    @pl.when(i == 0)
    def _():
        o_ref[...] = jnp.broadcast_to(b3_ref[...], o_ref.shape)

    h2 = jnp.dot(h1_ref[...], w2_ref[...], preferred_element_type=jnp.float32)
    h2 = jnp.maximum(h2 + b2_ref[...], 0.0)
    o_ref[...] += jnp.dot(h2, w3_ref[...], preferred_element_type=jnp.float32)


def _h2_logits(h1, W2, b2, W3p, b3p):
    M, H = h1.shape
    NL = W3p.shape[1]
    bn2 = 256
    Bn = H // bn2
    return pl.pallas_call(
        _h2_logits_kernel,
        grid=(Bn,),
        in_specs=[
            pl.BlockSpec((M, H), lambda i: (0, 0)),
            pl.BlockSpec((H, bn2), lambda i: (0, i)),
            pl.BlockSpec((1, bn2), lambda i: (0, i)),
            pl.BlockSpec((bn2, NL), lambda i: (i, 0)),
            pl.BlockSpec((1, NL), lambda i: (0, 0)),
        ],
        out_specs=pl.BlockSpec((M, NL), lambda i: (0, 0)),
        out_shape=jax.ShapeDtypeStruct((M, NL), jnp.float32),
        compiler_params=pltpu.CompilerParams(
            dimension_semantics=("arbitrary",)),
    )(h1, W2, b2.reshape(1, H), W3p, b3p.reshape(1, NL))


def _transpose_kernel(a_ref, o_ref):
    o_ref[...] = a_ref[...].T


def _transpose(a, bt=128):
    M, NL = a.shape
    return pl.pallas_call(
        _transpose_kernel,
        grid=(M // bt,),
        in_specs=[pl.BlockSpec((bt, NL), lambda i: (i, 0))],
        out_specs=pl.BlockSpec((NL, bt), lambda i: (0, i)),
        out_shape=jax.ShapeDtypeStruct((NL, M), jnp.float32),
    )(a)


def _sc_route(logitsT, E, capacity):
    NL, T = logitsT.shape
    NS = 16
    NT = T // NS
    G = NT // 16
    mesh = plsc.VectorSubcoreMesh(core_axis_name="c", subcore_axis_name="s")
    neginf = float("-inf")

    @functools.partial(
        pl.kernel,
        mesh=mesh,
        out_type=jax.ShapeDtypeStruct((T,), jnp.float32),
        scratch_types=[
            pltpu.VMEM((E, T), jnp.float32),
            pltpu.VMEM((NT,), jnp.float32),
            pltpu.VMEM((32,), jnp.float32),
        ],
    )
    def route(lg_hbm, gate_hbm, chunk, gates, buff):
        c = lax.axis_index("c")
        s = lax.axis_index("s")
        base = s * NT
        pltpu.sync_copy(lg_hbm.at[pl.ds(0, E), :], chunk)
        zf = jnp.zeros((16,), jnp.float32)
        onef = jnp.full((16,), 1.0, jnp.float32)
        ninf = jnp.full((16,), neginf, jnp.float32)
        buff[pl.ds(0, 16)] = zf

        def shrf(x, k):
            buff[pl.ds(16, 16)] = x
            return buff[pl.ds(16 - k, 16)]

        def top2(t0):
            l = [chunk[e, pl.ds(t0, 16)] for e in range(E)]
            m0 = l[0]
            for e in range(1, E):
                m0 = jnp.maximum(m0, l[e])
            e0 = jnp.full((16,), float(E), jnp.float32)
            for e in reversed(range(E)):
                e0 = jnp.where(l[e] == m0, float(e), e0)
            pm = [jnp.where(e0 == float(e), ninf, l[e]) for e in range(E)]
            m1 = pm[0]
            for e in range(1, E):
                m1 = jnp.maximum(m1, pm[e])
            e1 = jnp.full((16,), float(E), jnp.float32)
            for e in reversed(range(E)):
                e1 = jnp.where(pm[e] == m1, float(e), e1)
            return l, m0, m1, e0, e1

        def cbody(g, accs):
            l, m0, m1, e0, e1 = top2(g * 16)
            out = []
            for e in range(E):
                ef = float(e)
                ind = (jnp.where(e0 == ef, onef, zf)
                       + jnp.where(e1 == ef, onef, zf))
                out.append(accs[e] + ind)
            return tuple(out)

        accs = lax.fori_loop(0, s * G, cbody, (zf,) * E)
        cnt = []
        for e in range(E):
            t = accs[e]
            for k in (1, 2, 4, 8):
                t = t + shrf(t, k)
            cnt.append(t[15])

        capf = float(capacity)
        for g in range(G):
            sl = pl.ds(g * 16, 16)
            l, m0, m1, e0, e1 = top2(base + g * 16)
            ex = [jnp.exp(l[e] - m0) for e in range(E)]
            den = ex[0]
            for e in range(1, E):
                den = den + ex[e]
            s0 = onef / den
            s1 = jnp.exp(m1 - m0) / den
            pos0 = zf
            pos1 = zf
            for e in range(E):
                ef = float(e)
                i0 = jnp.where(e0 == ef, onef, zf)
                i1 = jnp.where(e1 == ef, onef, zf)
                ind = i0 + i1
                incl = ind
                for k in (1, 2, 4, 8):
                    incl = incl + shrf(incl, k)
                exc = incl - ind
                posadd = exc + (zf + cnt[e])
                pos0 = pos0 + i0 * posadd
                pos1 = pos1 + i1 * posadd
                cnt[e] = cnt[e] + incl[15]
            keep0 = jnp.where(pos0 < capf, onef, zf)
            keep1 = jnp.where(pos1 < capf, onef, zf)
            gates[sl] = keep0 * s0 + keep1 * s1

        @pl.when(c == 0)
        def _():
            pltpu.sync_copy(gates, gate_hbm.at[pl.ds(base, NT)])

    return route(logitsT)


def _ffn_kernel(x_ref, w1_ref, b1_ref, w2_ref, b2_ref, o_ref):
    y1 = jnp.dot(x_ref[...], w1_ref[...], preferred_element_type=jnp.float32)
    y1 = jnp.maximum(y1 + b1_ref[...], 0.0)
    y = jnp.dot(y1, w2_ref[...], preferred_element_type=jnp.float32)
    o_ref[...] = y + b2_ref[...]


def _ffn(x, We1, be1, We2, be2, bm):
    M, C = x.shape
    _, H = We1.shape
    _, N = We2.shape
    return pl.pallas_call(
        _ffn_kernel,
        grid=(M // bm,),
        in_specs=[
            pl.BlockSpec((bm, C), lambda m: (m, 0)),
            pl.BlockSpec((C, H), lambda m: (0, 0)),
            pl.BlockSpec((1, H), lambda m: (0, 0)),
            pl.BlockSpec((H, N), lambda m: (0, 0)),
            pl.BlockSpec((1, N), lambda m: (0, 0)),
        ],
        out_specs=pl.BlockSpec((bm, N), lambda m: (m, 0)),
        out_shape=jax.ShapeDtypeStruct((M, N), jnp.float32),
    )(x, We1, be1.reshape(1, H), We2, be2.reshape(1, N))


def _mul_gate_kernel(y_ref, g_ref, o_ref):
    o_ref[...] = y_ref[...] * g_ref[...]


def _mul_gate(y, gate, bm):
    M, N = y.shape
    return pl.pallas_call(
        _mul_gate_kernel,
        grid=(M // bm,),
        in_specs=[
            pl.BlockSpec((bm, N), lambda m: (m, 0)),
            pl.BlockSpec((bm, 1), lambda m: (m, 0)),
        ],
        out_specs=pl.BlockSpec((bm, N), lambda m: (m, 0)),
        out_shape=jax.ShapeDtypeStruct((M, N), jnp.float32),
    )(y, gate)


def kernel(x, W1, b1, W2, b2, W3, b3, We1, be1, We2, be2):
    B, T, C = x.shape
    E = W3.shape[1]
    capacity = int(T / E * 1.25)
    xf = x.reshape(T, C)

    W3p = jnp.pad(W3, ((0, 0), (0, 128 - E)))
    b3p = jnp.pad(b3, (0, 128 - E))
    h1 = _mm_bias(xf, W1, b1, relu=True, bm=2048, bn=512)
    logits = _h2_logits(h1, W2, b2, W3p, b3p)
    gate = _sc_route(_transpose(logits), E, capacity)
    y = _ffn(xf, We1, be1, We2, be2, bm=512)
    out = _mul_gate(y, gate.reshape(T, 1), bm=512)
    return out.reshape(B, T, C)


# speedup vs baseline: 1.0328x; 1.0328x over previous
import functools
import jax
import jax.numpy as jnp
from jax import lax
from jax.experimental import pallas as pl
from jax.experimental.pallas import tpu as pltpu
from jax.experimental.pallas import tpu_sc as plsc


def _mm_bias_kernel(a_ref, w_ref, b_ref, o_ref, *, relu):
    acc = jnp.dot(a_ref[...], w_ref[...], preferred_element_type=jnp.float32)
    acc = acc + b_ref[...]
    if relu:
        acc = jnp.maximum(acc, 0.0)
    o_ref[...] = acc


def _mm_bias(a, w, b, relu, bm, bn):
    M, K = a.shape
    _, N = w.shape
    return pl.pallas_call(
        functools.partial(_mm_bias_kernel, relu=relu),
        grid=(N // bn, M // bm),
        in_specs=[
            pl.BlockSpec((bm, K), lambda n, m: (m, 0)),
            pl.BlockSpec((K, bn), lambda n, m: (0, n)),
            pl.BlockSpec((1, bn), lambda n, m: (0, n)),
        ],
        out_specs=pl.BlockSpec((bm, bn), lambda n, m: (m, n)),
        out_shape=jax.ShapeDtypeStruct((M, N), jnp.float32),
    )(a, w, b.reshape(1, N))


def _h2_logits_kernel(h1_ref, w2_ref, b2_ref, w3_ref, b3_ref, o_ref):
    i = pl.program_id(0)

    @pl.when(i == 0)
    def _():
        o_ref[...] = jnp.broadcast_to(b3_ref[...], o_ref.shape)

    h2 = jnp.dot(h1_ref[...], w2_ref[...], preferred_element_type=jnp.float32)
    h2 = jnp.maximum(h2 + b2_ref[...], 0.0)
    o_ref[...] += jnp.dot(h2, w3_ref[...], preferred_element_type=jnp.float32)


def _h2_logits(h1, W2, b2, W3p, b3p):
    M, H = h1.shape
    NL = W3p.shape[1]
    bn2 = 256
    Bn = H // bn2
    return pl.pallas_call(
        _h2_logits_kernel,
        grid=(Bn,),
        in_specs=[
            pl.BlockSpec((M, H), lambda i: (0, 0)),
            pl.BlockSpec((H, bn2), lambda i: (0, i)),
            pl.BlockSpec((1, bn2), lambda i: (0, i)),
            pl.BlockSpec((bn2, NL), lambda i: (i, 0)),
            pl.BlockSpec((1, NL), lambda i: (0, 0)),
        ],
        out_specs=pl.BlockSpec((M, NL), lambda i: (0, 0)),
        out_shape=jax.ShapeDtypeStruct((M, NL), jnp.float32),
        compiler_params=pltpu.CompilerParams(
            dimension_semantics=("arbitrary",)),
    )(h1, W2, b2.reshape(1, H), W3p, b3p.reshape(1, NL))


def _transpose_kernel(a_ref, o_ref):
    o_ref[...] = a_ref[...].T


def _transpose(a, bt=128):
    M, NL = a.shape
    return pl.pallas_call(
        _transpose_kernel,
        grid=(M // bt,),
        in_specs=[pl.BlockSpec((bt, NL), lambda i: (i, 0))],
        out_specs=pl.BlockSpec((NL, bt), lambda i: (0, i)),
        out_shape=jax.ShapeDtypeStruct((NL, M), jnp.float32),
    )(a)


def _sc_route(logitsT, E, capacity):
    NL, T = logitsT.shape
    NS = 16
    NT = T // NS
    G = NT // 16
    mesh = plsc.VectorSubcoreMesh(core_axis_name="c", subcore_axis_name="s")
    neginf = float("-inf")

    @functools.partial(
        pl.kernel,
        mesh=mesh,
        out_type=jax.ShapeDtypeStruct((T,), jnp.float32),
        scratch_types=[
            pltpu.VMEM((E, NT), jnp.float32),
            pltpu.VMEM((NT,), jnp.float32),
            pltpu.VMEM((32,), jnp.float32),
            pltpu.VMEM((E * 16,), jnp.float32),
            pltpu.VMEM((NS * E * 16,), jnp.float32),
            pltpu.VMEM_SHARED((NS * E * 16,), jnp.float32),
        ],
    )
    def route(lg_hbm, gate_hbm, chunk, gates, buff, mycnt, allcnt, shared):
        c = lax.axis_index("c")
        s = lax.axis_index("s")
        base = s * NT
        pltpu.sync_copy(lg_hbm.at[pl.ds(0, E), pl.ds(base, NT)], chunk)
        zf = jnp.zeros((16,), jnp.float32)
        onef = jnp.full((16,), 1.0, jnp.float32)
        ninf = jnp.full((16,), neginf, jnp.float32)
        buff[pl.ds(0, 16)] = zf

        def shrf(x, k):
            buff[pl.ds(16, 16)] = x
            return buff[pl.ds(16 - k, 16)]

        def top2(t0):
            l = [chunk[e, pl.ds(t0, 16)] for e in range(E)]
            m0 = l[0]
            for e in range(1, E):
                m0 = jnp.maximum(m0, l[e])
            e0 = jnp.full((16,), float(E), jnp.float32)
            for e in reversed(range(E)):
                e0 = jnp.where(l[e] == m0, float(e), e0)
            pm = [jnp.where(e0 == float(e), ninf, l[e]) for e in range(E)]
            m1 = pm[0]
            for e in range(1, E):
                m1 = jnp.maximum(m1, pm[e])
            e1 = jnp.full((16,), float(E), jnp.float32)
            for e in reversed(range(E)):
                e1 = jnp.where(pm[e] == m1, float(e), e1)
            return l, m0, m1, e0, e1

        accs = [zf] * E
        for g in range(G):
            l, m0, m1, e0, e1 = top2(g * 16)
            for e in range(E):
                ef = float(e)
                accs[e] = (accs[e]
                           + jnp.where(e0 == ef, onef, zf)
                           + jnp.where(e1 == ef, onef, zf))
        for e in range(E):
            mycnt[pl.ds(e * 16, 16)] = accs[e]

        pltpu.sync_copy(mycnt, shared.at[pl.ds(s * (E * 16), E * 16)])
        plsc.subcore_barrier()
        pltpu.sync_copy(shared, allcnt)

        sv = zf + lax.convert_element_type(s, jnp.float32)
        cnt = []
        for e in range(E):
            t = zf
            for sub in range(NS):
                mask = jnp.where(jnp.full((16,), float(sub)) < sv, onef, zf)
                t = t + mask * allcnt[pl.ds(sub * (E * 16) + e * 16, 16)]
            for k in (1, 2, 4, 8):
                t = t + shrf(t, k)
            cnt.append(t[15])

        capf = float(capacity)
        for g in range(G):
            sl = pl.ds(g * 16, 16)
            l, m0, m1, e0, e1 = top2(g * 16)
            ex = [jnp.exp(l[e] - m0) for e in range(E)]
            den = ex[0]
            for e in range(1, E):
                den = den + ex[e]
            s0 = onef / den
            s1 = jnp.exp(m1 - m0) / den
            pos0 = zf
            pos1 = zf
            for e in range(E):
                ef = float(e)
                i0 = jnp.where(e0 == ef, onef, zf)
                i1 = jnp.where(e1 == ef, onef, zf)
                ind = i0 + i1
                incl = ind
                for k in (1, 2, 4, 8):
                    incl = incl + shrf(incl, k)
                exc = incl - ind
                posadd = exc + (zf + cnt[e])
                pos0 = pos0 + i0 * posadd
                pos1 = pos1 + i1 * posadd
                cnt[e] = cnt[e] + incl[15]
            keep0 = jnp.where(pos0 < capf, onef, zf)
            keep1 = jnp.where(pos1 < capf, onef, zf)
            gates[sl] = keep0 * s0 + keep1 * s1

        @pl.when(c == 0)
        def _():
            pltpu.sync_copy(gates, gate_hbm.at[pl.ds(base, NT)])

    return route(logitsT)


def _ffn_kernel(x_ref, w1_ref, b1_ref, w2_ref, b2_ref, o_ref):
    y1 = jnp.dot(x_ref[...], w1_ref[...], preferred_element_type=jnp.float32)
    y1 = jnp.maximum(y1 + b1_ref[...], 0.0)
    y = jnp.dot(y1, w2_ref[...], preferred_element_type=jnp.float32)
    o_ref[...] = y + b2_ref[...]


def _ffn(x, We1, be1, We2, be2, bm):
    M, C = x.shape
    _, H = We1.shape
    _, N = We2.shape
    return pl.pallas_call(
        _ffn_kernel,
        grid=(M // bm,),
        in_specs=[
            pl.BlockSpec((bm, C), lambda m: (m, 0)),
            pl.BlockSpec((C, H), lambda m: (0, 0)),
            pl.BlockSpec((1, H), lambda m: (0, 0)),
            pl.BlockSpec((H, N), lambda m: (0, 0)),
            pl.BlockSpec((1, N), lambda m: (0, 0)),
        ],
        out_specs=pl.BlockSpec((bm, N), lambda m: (m, 0)),
        out_shape=jax.ShapeDtypeStruct((M, N), jnp.float32),
    )(x, We1, be1.reshape(1, H), We2, be2.reshape(1, N))


def _mul_gate_kernel(y_ref, g_ref, o_ref):
    o_ref[...] = y_ref[...] * g_ref[...]


def _mul_gate(y, gate, bm):
    M, N = y.shape
    return pl.pallas_call(
        _mul_gate_kernel,
        grid=(M // bm,),
        in_specs=[
            pl.BlockSpec((bm, N), lambda m: (m, 0)),
            pl.BlockSpec((bm, 1), lambda m: (m, 0)),
        ],
        out_specs=pl.BlockSpec((bm, N), lambda m: (m, 0)),
        out_shape=jax.ShapeDtypeStruct((M, N), jnp.float32),
    )(y, gate)


def kernel(x, W1, b1, W2, b2, W3, b3, We1, be1, We2, be2):
    B, T, C = x.shape
    E = W3.shape[1]
    capacity = int(T / E * 1.25)
    xf = x.reshape(T, C)

    W3p = jnp.pad(W3, ((0, 0), (0, 128 - E)))
    b3p = jnp.pad(b3, (0, 128 - E))
    h1 = _mm_bias(xf, W1, b1, relu=True, bm=2048, bn=512)
    logits = _h2_logits(h1, W2, b2, W3p, b3p)
    gate = _sc_route(_transpose(logits), E, capacity)
    y = _ffn(xf, We1, be1, We2, be2, bm=512)
    out = _mul_gate(y, gate.reshape(T, 1), bm=512)
    return out.reshape(B, T, C)
